# baseline (speedup 1.0000x reference)
import functools

import jax
import jax.numpy as jnp
from jax import lax
from jax.experimental import pallas as pl
from jax.experimental.pallas import tpu as pltpu
from jax.experimental.pallas import tpu_sc as plsc

DICT_SIZE = 1000000
D = 64
B = 16384
CTX = 10
TGT = 6
L = 16
W = 128

NC = 2
NS = 16
NW = NC * NS
PW = B // NW
CB = 64
NCHUNK = PW // CB
CIDX_ROWS = CB * CTX // 128
TIDX_ROWS = CB * TGT // 128
CIDX_W = PW * CTX // 128
TIDX_W = PW * TGT // 128


def _pack_bf16_pairs(table):
    u = lax.bitcast_convert_type(table, jnp.int32)
    ue = u[0::2, :]
    uo = u[1::2, :]
    rnd_e = ue + jnp.int32(0x7FFF) + ((ue >> 16) & 1)
    rnd_o = uo + jnp.int32(0x7FFF) + ((uo >> 16) & 1)
    packed = ((rnd_e >> 16) & jnp.int32(0xFFFF)) | (rnd_o & jnp.int32(-65536))
    return packed.reshape(DICT_SIZE // 4, W)


def kernel(context_ids, target_ids, input_table, output_table):
    ctx_idx = context_ids.astype(jnp.int32).reshape(B * CTX // 128, 128)
    tgt_idx = target_ids.astype(jnp.int32).reshape(B * TGT // 128, 128)
    itab = _pack_bf16_pairs(input_table)
    otab = _pack_bf16_pairs(output_table)

    mesh = plsc.VectorSubcoreMesh(core_axis_name="c", subcore_axis_name="s")

    @functools.partial(
        pl.kernel,
        mesh=mesh,
        out_type=jax.ShapeDtypeStruct((B, L), jnp.float32),
        scratch_types=[
            pltpu.VMEM((CIDX_W, 128), jnp.int32),
            pltpu.VMEM((TIDX_W, 128), jnp.int32),
            pltpu.VMEM((CIDX_W, 128), jnp.int32),
            pltpu.VMEM((TIDX_W, 128), jnp.int32),
            pltpu.VMEM((CB * CTX, W), jnp.int32),
            pltpu.VMEM((CB, D), jnp.float32),
            pltpu.VMEM((CB, L), jnp.float32),
            pltpu.SemaphoreType.DMA,
        ],
    )
    def sc_kernel(ctx_hbm, tgt_hbm, itab_hbm, otab_hbm, out_hbm,
                  cidx_v, tidx_v, cpk_v, tpk_v, rows_v, mean_v, pad_v, sem):
        wid = lax.axis_index("s") * NC + lax.axis_index("c")
        lane = lax.broadcasted_iota(jnp.int32, (L,), 0)
        perms = [lane ^ 8, lane ^ 4, lane ^ 2, lane ^ 1]

        pltpu.sync_copy(ctx_hbm.at[pl.ds(wid * CIDX_W, CIDX_W)], cidx_v)
        pltpu.sync_copy(tgt_hbm.at[pl.ds(wid * TIDX_W, TIDX_W)], tidx_v)

        def pack_c(i, carry):
            for s in range(128 // L):
                cpk_v[i, pl.ds(s * L, L)] = cidx_v[i, pl.ds(s * L, L)] >> 2
            return carry

        def pack_t(i, carry):
            for s in range(128 // L):
                tpk_v[i, pl.ds(s * L, L)] = tidx_v[i, pl.ds(s * L, L)] >> 2
            return carry

        lax.fori_loop(0, CIDX_W, pack_c, 0)
        lax.fori_loop(0, TIDX_W, pack_t, 0)

        def row_vals(idx_ref, fp, r):
            idv = idx_ref[fp // 128, pl.ds((fp % 128) & ~15, L)]
            pid = jnp.take(idv, jnp.full((L,), fp & 15, jnp.int32))
            sh0 = (pid & 1) * L
            s1 = (pid >> 1) & 1
            out = []
            for k in range(D // L):
                a = rows_v[r, pl.ds(k * L, L)]
                b = rows_v[r, pl.ds(D + k * L, L)]
                ui = a + s1 * (b - a)
                out.append(lax.bitcast_convert_type(
                    lax.shift_right_logical(ui, sh0) << 16, jnp.float32))
            return out

        for c in range(NCHUNK):
            chunk = wid * NCHUNK + c
            copies = []
            for j in range(CIDX_ROWS):
                copies.append(pltpu.async_copy(
                    itab_hbm.at[cpk_v.at[c * CIDX_ROWS + j]],
                    rows_v.at[pl.ds(j * 128, 128)], sem))
            for cp in copies:
                cp.wait()

            def mean_body(b, carry):
                accs = [None] * (D // L)
                for j in range(CTX):
                    r = b * CTX + j
                    vs = row_vals(cidx_v, c * CB * CTX + r, r)
                    for k in range(D // L):
                        accs[k] = (vs[k] if accs[k] is None
                                   else accs[k] + vs[k])
                for k in range(D // L):
                    mean_v[b, pl.ds(k * L, L)] = accs[k] * (1.0 / CTX)
                return carry

            lax.fori_loop(0, CB, mean_body, 0)

            copies = []
            for j in range(TIDX_ROWS):
                copies.append(pltpu.async_copy(
                    otab_hbm.at[tpk_v.at[c * TIDX_ROWS + j]],
                    rows_v.at[pl.ds(j * 128, 128)], sem))
            for cp in copies:
                cp.wait()

            def dot_body(b, carry):
                ms = [mean_v[b, pl.ds(k * L, L)] for k in range(D // L)]
                logit = jnp.zeros((L,), jnp.float32)
                for t in range(TGT):
                    r = b * TGT + t
                    vs = row_vals(tidx_v, c * CB * TGT + r, r)
                    p = None
                    for k in range(D // L):
                        pk = ms[k] * vs[k]
                        p = pk if p is None else p + pk
                    for pm in perms:
                        p = p + jnp.take(p, pm)
                    logit = jnp.where(lane == t, p, logit)
                pad_v[b] = 1.0 / (1.0 + jnp.exp(-logit))
                return carry

            lax.fori_loop(0, CB, dot_body, 0)
            pltpu.sync_copy(pad_v, out_hbm.at[pl.ds(chunk * CB, CB)])

    out = sc_kernel(ctx_idx, tgt_idx, itab, otab)
    return out[:, :TGT]


# speedup vs baseline: 15.4731x; 15.4731x over previous
import functools

import jax
import jax.numpy as jnp
from jax import lax
from jax.experimental import layout as jlayout
from jax.experimental import pallas as pl
from jax.experimental.pallas import tpu as pltpu
from jax.experimental.pallas import tpu_sc as plsc

DICT_SIZE = 1000000
D = 64
B = 16384
CTX = 10
TGT = 6
L = 16
W = 128

NC = 2
NS = 16
NW = NC * NS
PW = B // NW
CB = 64
NCHUNK = PW // CB
CIDX_ROWS = CB * CTX // 128
TIDX_ROWS = CB * TGT // 128
CIDX_W = PW * CTX // 128
TIDX_W = PW * TGT // 128


def _pad_rows(table):
    padded = jnp.pad(table, ((0, 0), (0, W - D)))
    return jlayout.with_layout_constraint(
        padded, jlayout.Layout(major_to_minor=(0, 1)))


def kernel(context_ids, target_ids, input_table, output_table):
    ctx_idx = context_ids.astype(jnp.int32).reshape(B * CTX // 128, 128)
    tgt_idx = target_ids.astype(jnp.int32).reshape(B * TGT // 128, 128)
    itab = _pad_rows(input_table)
    otab = _pad_rows(output_table)

    mesh = plsc.VectorSubcoreMesh(core_axis_name="c", subcore_axis_name="s")

    @functools.partial(
        pl.kernel,
        mesh=mesh,
        out_type=jax.ShapeDtypeStruct((B, L), jnp.float32),
        scratch_types=[
            pltpu.VMEM((CIDX_W, 128), jnp.int32),
            pltpu.VMEM((TIDX_W, 128), jnp.int32),
            pltpu.VMEM((CB * CTX, W), jnp.float32),
            pltpu.VMEM((CB, D), jnp.float32),
            pltpu.VMEM((CB, L), jnp.float32),
            pltpu.SemaphoreType.DMA,
        ],
    )
    def sc_kernel(ctx_hbm, tgt_hbm, itab_hbm, otab_hbm, out_hbm,
                  cidx_v, tidx_v, rows_v, mean_v, pad_v, sem):
        wid = lax.axis_index("s") * NC + lax.axis_index("c")
        lane = lax.broadcasted_iota(jnp.int32, (L,), 0)
        perms = [lane ^ 8, lane ^ 4, lane ^ 2, lane ^ 1]

        pltpu.sync_copy(ctx_hbm.at[pl.ds(wid * CIDX_W, CIDX_W)], cidx_v)
        pltpu.sync_copy(tgt_hbm.at[pl.ds(wid * TIDX_W, TIDX_W)], tidx_v)

        for c in range(NCHUNK):
            chunk = wid * NCHUNK + c
            copies = []
            for j in range(CIDX_ROWS):
                copies.append(pltpu.async_copy(
                    itab_hbm.at[cidx_v.at[c * CIDX_ROWS + j]],
                    rows_v.at[pl.ds(j * 128, 128)], sem))
            for cp in copies:
                cp.wait()

            def mean_body(b, carry):
                accs = [None] * (D // L)
                for j in range(CTX):
                    r = b * CTX + j
                    for k in range(D // L):
                        v = rows_v[r, pl.ds(k * L, L)]
                        accs[k] = (v if accs[k] is None else accs[k] + v)
                for k in range(D // L):
                    mean_v[b, pl.ds(k * L, L)] = accs[k] * (1.0 / CTX)
                return carry

            lax.fori_loop(0, CB, mean_body, 0)

            copies = []
            for j in range(TIDX_ROWS):
                copies.append(pltpu.async_copy(
                    otab_hbm.at[tidx_v.at[c * TIDX_ROWS + j]],
                    rows_v.at[pl.ds(j * 128, 128)], sem))
            for cp in copies:
                cp.wait()

            def dot_body(b, carry):
                ms = [mean_v[b, pl.ds(k * L, L)] for k in range(D // L)]
                logit = jnp.zeros((L,), jnp.float32)
                for t in range(TGT):
                    r = b * TGT + t
                    p = None
                    for k in range(D // L):
                        pk = ms[k] * rows_v[r, pl.ds(k * L, L)]
                        p = pk if p is None else p + pk
                    for pm in perms:
                        p = p + jnp.take(p, pm)
                    logit = jnp.where(lane == t, p, logit)
                pad_v[b] = 1.0 / (1.0 + jnp.exp(-logit))
                return carry

            lax.fori_loop(0, CB, dot_body, 0)
            pltpu.sync_copy(pad_v, out_hbm.at[pl.ds(chunk * CB, CB)])

    out = sc_kernel(ctx_idx, tgt_idx, itab, otab)
    return out[:, :TGT]


# speedup vs baseline: 16.0074x; 1.0345x over previous
import functools

import jax
import jax.numpy as jnp
from jax import lax
from jax.experimental import pallas as pl
from jax.experimental.pallas import tpu as pltpu
from jax.experimental.pallas import tpu_sc as plsc

DICT_SIZE = 1000000
D = 64
B = 16384
CTX = 10
TGT = 6
L = 16
W = 128

NC = 2
NS = 16
NW = NC * NS
PW = B // NW
CB = 64
NCHUNK = PW // CB
CIDX_ROWS = CB * CTX // 128
TIDX_ROWS = CB * TGT // 128
CIDX_W = PW * CTX // 128
TIDX_W = PW * TGT // 128


def kernel(context_ids, target_ids, input_table, output_table):
    ctx_idx = context_ids.astype(jnp.int32).reshape(B * CTX // 128, 128)
    tgt_idx = target_ids.astype(jnp.int32).reshape(B * TGT // 128, 128)
    itab = jnp.pad(input_table, ((0, 0), (0, W - D)))
    otab = jnp.pad(output_table, ((0, 0), (0, W - D)))

    mesh = plsc.VectorSubcoreMesh(core_axis_name="c", subcore_axis_name="s")

    @functools.partial(
        pl.kernel,
        mesh=mesh,
        out_type=jax.ShapeDtypeStruct((B, L), jnp.float32),
        scratch_types=[
            pltpu.VMEM((CIDX_W, 128), jnp.int32),
            pltpu.VMEM((TIDX_W, 128), jnp.int32),
            pltpu.VMEM((CB * CTX, W), jnp.float32),
            pltpu.VMEM((CB, D), jnp.float32),
            pltpu.VMEM((CB, L), jnp.float32),
            pltpu.SemaphoreType.DMA,
        ],
    )
    def sc_kernel(ctx_hbm, tgt_hbm, itab_hbm, otab_hbm, out_hbm,
                  cidx_v, tidx_v, rows_v, mean_v, pad_v, sem):
        wid = lax.axis_index("s") * NC + lax.axis_index("c")
        lane = lax.broadcasted_iota(jnp.int32, (L,), 0)
        perms = [lane ^ 8, lane ^ 4, lane ^ 2, lane ^ 1]

        pltpu.sync_copy(ctx_hbm.at[pl.ds(wid * CIDX_W, CIDX_W)], cidx_v)
        pltpu.sync_copy(tgt_hbm.at[pl.ds(wid * TIDX_W, TIDX_W)], tidx_v)

        for c in range(NCHUNK):
            chunk = wid * NCHUNK + c
            copies = []
            for j in range(CIDX_ROWS):
                copies.append(pltpu.async_copy(
                    itab_hbm.at[cidx_v.at[c * CIDX_ROWS + j]],
                    rows_v.at[pl.ds(j * 128, 128)], sem))
            for cp in copies:
                cp.wait()

            def mean_body(b, carry):
                accs = [None] * (D // L)
                for j in range(CTX):
                    r = b * CTX + j
                    for k in range(D // L):
                        v = rows_v[r, pl.ds(k * L, L)]
                        accs[k] = (v if accs[k] is None else accs[k] + v)
                for k in range(D // L):
                    mean_v[b, pl.ds(k * L, L)] = accs[k] * (1.0 / CTX)
                return carry

            lax.fori_loop(0, CB, mean_body, 0)

            copies = []
            for j in range(TIDX_ROWS):
                copies.append(pltpu.async_copy(
                    otab_hbm.at[tidx_v.at[c * TIDX_ROWS + j]],
                    rows_v.at[pl.ds(j * 128, 128)], sem))
            for cp in copies:
                cp.wait()

            def dot_body(b, carry):
                ms = [mean_v[b, pl.ds(k * L, L)] for k in range(D // L)]
                logit = jnp.zeros((L,), jnp.float32)
                for t in range(TGT):
                    r = b * TGT + t
                    p = None
                    for k in range(D // L):
                        pk = ms[k] * rows_v[r, pl.ds(k * L, L)]
                        p = pk if p is None else p + pk
                    for pm in perms:
                        p = p + jnp.take(p, pm)
                    logit = jnp.where(lane == t, p, logit)
                pad_v[b] = 1.0 / (1.0 + jnp.exp(-logit))
                return carry

            lax.fori_loop(0, CB, dot_body, 0)
            pltpu.sync_copy(pad_v, out_hbm.at[pl.ds(chunk * CB, CB)])

    out = sc_kernel(ctx_idx, tgt_idx, itab, otab)
    return out[:, :TGT]


# speedup vs baseline: 16.1120x; 1.0065x over previous
import functools

import jax
import jax.numpy as jnp
from jax import lax
from jax.experimental import pallas as pl
from jax.experimental.pallas import tpu as pltpu
from jax.experimental.pallas import tpu_sc as plsc

DICT_SIZE = 1000000
D = 64
B = 16384
CTX = 10
TGT = 6
L = 16
W = 128

NC = 2
NS = 16
NW = NC * NS
PW = B // NW
CB = 64
NCHUNK = PW // CB
CIDX_ROWS = CB * CTX // 128
TIDX_ROWS = CB * TGT // 128
CIDX_W = PW * CTX // 128
TIDX_W = PW * TGT // 128


def kernel(context_ids, target_ids, input_table, output_table):
    ctx_idx = context_ids.astype(jnp.int32).reshape(B * CTX // 128, 128)
    tgt_idx = target_ids.astype(jnp.int32).reshape(B * TGT // 128, 128)
    itab = jnp.pad(input_table, ((0, 0), (0, W - D)))
    otab = jnp.pad(output_table, ((0, 0), (0, W - D)))

    mesh = plsc.VectorSubcoreMesh(core_axis_name="c", subcore_axis_name="s")

    @functools.partial(
        pl.kernel,
        mesh=mesh,
        out_type=jax.ShapeDtypeStruct((B, D), jnp.float32),
        scratch_types=[
            pltpu.VMEM((CIDX_W, 128), jnp.int32),
            pltpu.VMEM((CB * CTX, W), jnp.float32),
            pltpu.VMEM((CB, D), jnp.float32),
            pltpu.SemaphoreType.DMA,
        ],
    )
    def sc_mean(ctx_hbm, itab_hbm, mean_hbm, cidx_v, rows_v, mean_v, sem):
        wid = lax.axis_index("s") * NC + lax.axis_index("c")
        pltpu.sync_copy(ctx_hbm.at[pl.ds(wid * CIDX_W, CIDX_W)], cidx_v)
        for c in range(NCHUNK):
            chunk = wid * NCHUNK + c
            copies = []
            for j in range(CIDX_ROWS):
                copies.append(pltpu.async_copy(
                    itab_hbm.at[cidx_v.at[c * CIDX_ROWS + j]],
                    rows_v.at[pl.ds(j * 128, 128)], sem))
            for cp in copies:
                cp.wait()

            def mean_body(b, carry):
                accs = [None] * (D // L)
                for j in range(CTX):
                    r = b * CTX + j
                    for k in range(D // L):
                        v = rows_v[r, pl.ds(k * L, L)]
                        accs[k] = (v if accs[k] is None else accs[k] + v)
                for k in range(D // L):
                    mean_v[b, pl.ds(k * L, L)] = accs[k] * (1.0 / CTX)
                return carry

            lax.fori_loop(0, CB, mean_body, 0)
            pltpu.sync_copy(mean_v, mean_hbm.at[pl.ds(chunk * CB, CB)])

    @functools.partial(
        pl.kernel,
        mesh=mesh,
        out_type=jax.ShapeDtypeStruct((B, L), jnp.float32),
        scratch_types=[
            pltpu.VMEM((TIDX_W, 128), jnp.int32),
            pltpu.VMEM((CB * TGT, W), jnp.float32),
            pltpu.VMEM((CB, D), jnp.float32),
            pltpu.VMEM((CB, L), jnp.float32),
            pltpu.SemaphoreType.DMA,
        ],
    )
    def sc_dot(tgt_hbm, otab_hbm, mean_hbm, out_hbm,
               tidx_v, rows_v, mean_v, pad_v, sem):
        wid = lax.axis_index("s") * NC + lax.axis_index("c")
        lane = lax.broadcasted_iota(jnp.int32, (L,), 0)
        perms = [lane ^ 8, lane ^ 4, lane ^ 2, lane ^ 1]
        pltpu.sync_copy(tgt_hbm.at[pl.ds(wid * TIDX_W, TIDX_W)], tidx_v)
        for c in range(NCHUNK):
            chunk = wid * NCHUNK + c
            copies = [pltpu.async_copy(
                otab_hbm.at[tidx_v.at[c * TIDX_ROWS + j]],
                rows_v.at[pl.ds(j * 128, 128)], sem)
                for j in range(TIDX_ROWS)]
            pltpu.sync_copy(mean_hbm.at[pl.ds(chunk * CB, CB)], mean_v)
            for cp in copies:
                cp.wait()

            def dot_body(b, carry):
                ms = [mean_v[b, pl.ds(k * L, L)] for k in range(D // L)]
                logit = jnp.zeros((L,), jnp.float32)
                for t in range(TGT):
                    r = b * TGT + t
                    p = None
                    for k in range(D // L):
                        pk = ms[k] * rows_v[r, pl.ds(k * L, L)]
                        p = pk if p is None else p + pk
                    for pm in perms:
                        p = p + jnp.take(p, pm)
                    logit = jnp.where(lane == t, p, logit)
                pad_v[b] = 1.0 / (1.0 + jnp.exp(-logit))
                return carry

            lax.fori_loop(0, CB, dot_body, 0)
            pltpu.sync_copy(pad_v, out_hbm.at[pl.ds(chunk * CB, CB)])

    means = sc_mean(ctx_idx, itab)
    out = sc_dot(tgt_idx, otab, means)
    return out[:, :TGT]
